# baseline (speedup 1.0000x reference)
import functools
import math

import jax
import jax.numpy as jnp
from jax import lax
from jax.experimental import pallas as pl
from jax.experimental.pallas import tpu as pltpu
from jax.experimental.pallas import tpu_sc as plsc

D = 64
DP = 128
SCALE = math.sqrt(D)
V = 1000000

NC = 2
NS = 16
NW = NC * NS

X_ROWS = 4096
X_COLS = 200
GW = 128
NBUF_G = 4
NBUF_O = 4
G_STEPS = X_COLS


def _gather_body(xt_hbm, tp_hbm, out_hbm, idx_v,
                 g0, g1, g2, g3, o0, o1, o2, o3,
                 in_s0, in_s1, in_s2, in_s3,
                 out_s0, out_s1, out_s2, out_s3):
    gbufs = [g0, g1, g2, g3]
    obufs = [o0, o1, o2, o3]
    in_sems = [in_s0, in_s1, in_s2, in_s3]
    out_sems = [out_s0, out_s1, out_s2, out_s3]

    wid = lax.axis_index("s") * NC + lax.axis_index("c")
    i0w = wid * GW

    pltpu.sync_copy(xt_hbm.at[:, pl.ds(i0w, GW)], idx_v)

    def fire_gather(j, b):
        pltpu.async_copy(tp_hbm.at[idx_v.at[j]], gbufs[b], in_sems[b])

    def wait_gather(b):
        pltpu.make_async_copy(tp_hbm.at[pl.ds(0, GW)], gbufs[b],
                              in_sems[b]).wait()

    row_vecs = [i0 + lax.iota(jnp.int32, 16) for i0 in range(0, GW, 16)]

    def tr_scale(b, ob):
        gbuf, obuf = gbufs[b], obufs[ob]

        @plsc.parallel_loop(0, D, 1, unroll=4)
        def _(d):
            tr = lax.shift_right_logical(d, 3)
            r = lax.bitwise_and(d, 7)
            col = jnp.full((16,), d, jnp.int32)
            for k in range(GW // 16):
                obuf[tr, r, pl.ds(k * 16, 16)] = (
                    plsc.load_gather(gbuf, [row_vecs[k], col]) * SCALE)

    def fire_write(j, ob):
        pltpu.async_copy(obufs[ob], out_hbm.at[j, :, wid, :, :],
                         out_sems[ob])

    def wait_write(ob):
        pltpu.make_async_copy(obufs[ob], out_hbm.at[0, :, 0, :, :],
                              out_sems[ob]).wait()

    for b in range(NBUF_G):
        fire_gather(b, b)

    for b in range(NBUF_G):
        wait_gather(b)
        tr_scale(b, b % NBUF_O)
        fire_write(b, b % NBUF_O)
        fire_gather(b + NBUF_G, b)

    def outer(g, _):
        for b in range(NBUF_G):
            j = g * NBUF_G + b
            wait_gather(b)
            wait_write(b % NBUF_O)
            tr_scale(b, b % NBUF_O)
            fire_write(j, b % NBUF_O)
            fire_gather(j + NBUF_G, b)
        return 0

    lax.fori_loop(1, G_STEPS // NBUF_G - 1, outer, 0)

    for b in range(NBUF_G):
        j = (G_STEPS // NBUF_G - 1) * NBUF_G + b
        wait_gather(b)
        wait_write(b % NBUF_O)
        tr_scale(b, b % NBUF_O)
        fire_write(j, b % NBUF_O)

    for ob in range(NBUF_O):
        wait_write(ob)


def _gather(x_t, tpad):
    mesh = plsc.VectorSubcoreMesh(core_axis_name="c", subcore_axis_name="s")
    k = functools.partial(
        pl.kernel,
        mesh=mesh,
        out_type=jax.ShapeDtypeStruct((X_COLS, 8, NW, 8, 128), jnp.float32),
        scratch_types=(
            [pltpu.VMEM((X_COLS, GW), jnp.int32)]
            + [pltpu.VMEM((GW, DP), jnp.float32)] * NBUF_G
            + [pltpu.VMEM((8, 8, 128), jnp.float32)] * NBUF_O
            + [pltpu.SemaphoreType.DMA] * (NBUF_G + NBUF_O)
        ),
        compiler_params=pltpu.CompilerParams(use_tc_tiling_on_sc=False,
                                             needs_layout_passes=False),
    )(_gather_body)
    return k(x_t, tpad)


def kernel(x, table):
    x_t = x.astype(jnp.int32).T
    tpad = jnp.pad(table, ((0, 0), (0, DP - D)))
    out5 = _gather(x_t, tpad)
    out = out5.transpose(0, 1, 3, 2, 4).reshape(X_COLS, D, X_ROWS)
    return out.transpose(2, 0, 1)
